# baseline (speedup 1.0000x reference)
import functools

import jax
import jax.numpy as jnp
from jax import lax
from jax.experimental import pallas as pl
from jax.experimental.pallas import tpu as pltpu
from jax.experimental.pallas import tpu_sc as plsc

E = 320000
H = 128
D0, D1, D2 = 5, 6, 2
NT = D0 * D1 * D2
NTP = 64
NC, NS, L = 2, 16, 16
NW = NC * NS
U = 128
NU = E // U
GROUP = 6
NJ_MAIN = 78
NU_TAIL = NU - NW * NJ_MAIN


def _sc_body(codes_hbm, w0_hbm, w1_hbm, w2_hbm, out_hbm,
             w0_v, w1_v, w2_v, t_v, t_sh, code_v, rows_v,
             gsem, ssem, csem):
    c = lax.axis_index("c")
    s = lax.axis_index("s")
    wid = s * NC + c

    @pl.when(s == 0)
    def _build():
        pltpu.sync_copy(w0_hbm, w0_v)
        pltpu.sync_copy(w1_hbm, w1_v)
        pltpu.sync_copy(w2_hbm, w2_v)
        third = jnp.float32(1.0 / 3.0)
        for i0 in range(D0):
            for i1 in range(D1):
                for j in range(H // L):
                    sl = pl.ds(j * L, L)
                    s01 = w0_v[i0, sl] + w1_v[i1, sl]
                    r = (i0 * D1 + i1) * D2
                    t_v[r, sl] = (s01 + w2_v[0, sl]) * third
                    t_v[r + 1, sl] = (s01 + w2_v[1, sl]) * third
        pltpu.sync_copy(t_v, t_sh)

    plsc.subcore_barrier()

    u_base = wid * NJ_MAIN

    GU = GROUP * U

    def start_codes(u0, half, par):
        pltpu.async_copy(codes_hbm.at[pl.ds(u0 * U, GU)],
                         code_v.at[pl.ds(half, GU)], csem.at[par])

    def wait_codes(par):
        pltpu.make_async_copy(codes_hbm.at[pl.ds(0, GU)],
                              code_v.at[pl.ds(0, GU)], csem.at[par]).wait()

    start_codes(u_base, 0, 0)

    @pl.loop(0, NJ_MAIN, step=GROUP)
    def _group(g0):
        base_e = (u_base + g0) * U
        par = lax.rem(lax.div(g0, jnp.int32(GROUP)), jnp.int32(2))
        half = pl.multiple_of(par * GU, 8)
        wait_codes(par)

        @pl.when(g0 + GROUP < NJ_MAIN)
        def _prefetch():
            nxt = pl.multiple_of((1 - par) * GU, 8)
            start_codes(u_base + g0 + GROUP, nxt, 1 - par)

        CH = GROUP // 2
        gathers = []
        for p in range(2):
            @pl.when(g0 > 0)
            def _drain():
                pltpu.make_async_copy(
                    rows_v.at[p], out_hbm.at[pl.ds(base_e, CH * U)],
                    ssem.at[p]).wait()
            for h in range(CH):
                b = CH * p + h
                gathers.append(pltpu.async_copy(
                    t_sh.at[code_v.at[pl.ds(half + b * U, U)]],
                    rows_v.at[p].at[pl.ds(h * U, U)], gsem.at[b]))
        for p in range(2):
            for h in range(CH):
                gathers[CH * p + h].wait()
            pltpu.async_copy(
                rows_v.at[p], out_hbm.at[pl.ds(base_e + p * CH * U, CH * U)],
                ssem.at[p])

    CH = GROUP // 2
    for p in range(2):
        pltpu.make_async_copy(
            rows_v.at[p], out_hbm.at[pl.ds(0, CH * U)], ssem.at[p]).wait()

    @pl.when(wid < NU_TAIL)
    def _tail():
        u0 = NW * NJ_MAIN + wid
        base_e = u0 * U
        pltpu.sync_copy(codes_hbm.at[pl.ds(u0 * U, U)],
                        code_v.at[pl.ds(0, U)])
        pltpu.async_copy(t_sh.at[code_v.at[pl.ds(0, U)]],
                         rows_v.at[0].at[pl.ds(0, U)], gsem.at[0]).wait()
        pltpu.sync_copy(rows_v.at[0].at[pl.ds(0, U)],
                        out_hbm.at[pl.ds(base_e, U)])


_launch = functools.partial(
    pl.kernel,
    out_type=jax.ShapeDtypeStruct((E, H), jnp.float32),
    mesh=plsc.VectorSubcoreMesh(core_axis_name="c", subcore_axis_name="s"),
    scratch_types=[
        pltpu.VMEM((D0, H), jnp.float32),
        pltpu.VMEM((D1, H), jnp.float32),
        pltpu.VMEM((D2, H), jnp.float32),
        pltpu.VMEM((NTP, H), jnp.float32),
        pltpu.VMEM_SHARED((NTP, H), jnp.float32),
        pltpu.VMEM((2 * GROUP * U,), jnp.int32),
        pltpu.VMEM((2, (GROUP // 2) * U, H), jnp.float32),
        pltpu.SemaphoreType.DMA((GROUP,)),
        pltpu.SemaphoreType.DMA((2,)),
        pltpu.SemaphoreType.DMA((2,)),
    ],
)(_sc_body)


@jax.jit
def kernel(edge_attr, W0, W1, W2):
    codes = (edge_attr[:, 0] * (D1 * D2) + edge_attr[:, 1] * D2
             + edge_attr[:, 2])
    return _launch(codes, W0, W1, W2)


# speedup vs baseline: 1.0425x; 1.0425x over previous
import functools

import jax
import jax.numpy as jnp
from jax import lax
from jax.experimental import pallas as pl
from jax.experimental.pallas import tpu as pltpu
from jax.experimental.pallas import tpu_sc as plsc

E = 320000
H = 128
D0, D1, D2 = 5, 6, 2
NT = D0 * D1 * D2
NTP = 64
NC, NS, L = 2, 16, 16
NW = NC * NS
U = 128
NU = E // U
GROUP = 6
NJ_MAIN = 78
NU_TAIL = NU - NW * NJ_MAIN


def _sc_body(codes_hbm, w0_hbm, w1_hbm, w2_hbm, out_hbm,
             w0_v, w1_v, w2_v, t_v, t_sh, code_v, rows_v,
             gsem, ssem, csem):
    c = lax.axis_index("c")
    s = lax.axis_index("s")
    wid = s * NC + c

    NJU = NJ_MAIN * U
    codes_copy = pltpu.async_copy(
        codes_hbm.at[pl.ds(wid * NJU, NJU)],
        code_v.at[pl.ds(0, NJU)], csem.at[0])

    @pl.when(s == 0)
    def _build():
        pltpu.sync_copy(w0_hbm, w0_v)
        pltpu.sync_copy(w1_hbm, w1_v)
        pltpu.sync_copy(w2_hbm, w2_v)
        third = jnp.float32(1.0 / 3.0)
        for i0 in range(D0):
            for i1 in range(D1):
                for j in range(H // L):
                    sl = pl.ds(j * L, L)
                    s01 = w0_v[i0, sl] + w1_v[i1, sl]
                    r = (i0 * D1 + i1) * D2
                    t_v[r, sl] = (s01 + w2_v[0, sl]) * third
                    t_v[r + 1, sl] = (s01 + w2_v[1, sl]) * third
        pltpu.sync_copy(t_v, t_sh)

    plsc.subcore_barrier()

    u_base = wid * NJ_MAIN

    codes_copy.wait()

    @pl.loop(0, NJ_MAIN, step=GROUP)
    def _group(g0):
        base_e = (u_base + g0) * U
        half = pl.multiple_of(g0 * U, 8)
        gathers = []
        for p in range(GROUP // 2):
            @pl.when(g0 > 0)
            def _drain():
                pltpu.make_async_copy(
                    rows_v.at[p], out_hbm.at[pl.ds(base_e, 2 * U)],
                    ssem.at[p]).wait()
            for h in range(2):
                b = 2 * p + h
                gathers.append(pltpu.async_copy(
                    t_sh.at[code_v.at[pl.ds(half + b * U, U)]],
                    rows_v.at[p].at[pl.ds(h * U, U)], gsem.at[b]))
        for p in range(GROUP // 2):
            gathers[2 * p].wait()
            gathers[2 * p + 1].wait()
            pltpu.async_copy(
                rows_v.at[p], out_hbm.at[pl.ds(base_e + 2 * p * U, 2 * U)],
                ssem.at[p])

    for p in range(GROUP // 2):
        pltpu.make_async_copy(
            rows_v.at[p], out_hbm.at[pl.ds(0, 2 * U)], ssem.at[p]).wait()

    @pl.when(wid < NU_TAIL)
    def _tail():
        u0 = NW * NJ_MAIN + wid
        base_e = u0 * U
        pltpu.sync_copy(codes_hbm.at[pl.ds(u0 * U, U)],
                        code_v.at[pl.ds(0, U)])
        pltpu.async_copy(t_sh.at[code_v.at[pl.ds(0, U)]],
                         rows_v.at[0].at[pl.ds(0, U)], gsem.at[0]).wait()
        pltpu.sync_copy(rows_v.at[0].at[pl.ds(0, U)],
                        out_hbm.at[pl.ds(base_e, U)])


_launch = functools.partial(
    pl.kernel,
    out_type=jax.ShapeDtypeStruct((E, H), jnp.float32),
    mesh=plsc.VectorSubcoreMesh(core_axis_name="c", subcore_axis_name="s"),
    scratch_types=[
        pltpu.VMEM((D0, H), jnp.float32),
        pltpu.VMEM((D1, H), jnp.float32),
        pltpu.VMEM((D2, H), jnp.float32),
        pltpu.VMEM((NTP, H), jnp.float32),
        pltpu.VMEM_SHARED((NTP, H), jnp.float32),
        pltpu.VMEM((NJ_MAIN * U,), jnp.int32),
        pltpu.VMEM((GROUP // 2, 2 * U, H), jnp.float32),
        pltpu.SemaphoreType.DMA((GROUP,)),
        pltpu.SemaphoreType.DMA((GROUP // 2,)),
        pltpu.SemaphoreType.DMA((1,)),
    ],
)(_sc_body)


@jax.jit
def kernel(edge_attr, W0, W1, W2):
    codes = (edge_attr[:, 0] * (D1 * D2) + edge_attr[:, 1] * D2
             + edge_attr[:, 2])
    return _launch(codes, W0, W1, W2)
